# baseline (speedup 1.0000x reference)
import functools

import jax
import jax.numpy as jnp
from jax import lax
from jax.experimental import pallas as pl
from jax.experimental.pallas import tpu as pltpu
from jax.experimental.pallas import tpu_sc as plsc


@functools.cache
def _make_sc_gather(V, H, B):
    info = plsc.get_sparse_core_info()
    NC, NS = info.num_cores, info.num_subcores
    NW = NC * NS
    assert B % (8 * NW) == 0
    b_per_w = B // NW
    mesh = plsc.VectorSubcoreMesh(core_axis_name="c", subcore_axis_name="s")

    @functools.partial(
        pl.kernel,
        mesh=mesh,
        out_type=jax.ShapeDtypeStruct((B, H), jnp.float32),
        scratch_types=[
            pltpu.VMEM((b_per_w,), jnp.int32),
            pltpu.VMEM((b_per_w, H), jnp.float32),
            pltpu.SemaphoreType.DMA,
        ],
    )
    def gather_k(table_hbm, idx_hbm, out_hbm, idx_v, rows_v, sem):
        wid = lax.axis_index("s") * NC + lax.axis_index("c")
        base = wid * b_per_w
        pltpu.sync_copy(idx_hbm.at[pl.ds(base, b_per_w)], idx_v)
        for c in range(b_per_w // 16):
            chunk = idx_v[pl.ds(c * 16, 16)]
            for i in range(16):
                pltpu.async_copy(
                    table_hbm.at[pl.ds(chunk[i], 1)],
                    rows_v.at[pl.ds(c * 16 + i, 1)],
                    sem,
                )
        pltpu.make_async_copy(
            table_hbm.at[pl.ds(0, b_per_w)], rows_v, sem
        ).wait()
        pltpu.sync_copy(rows_v, out_hbm.at[pl.ds(base, b_per_w)])

    return gather_k


def _proj_body(x_ref, wt_ref, b_ref, o_ref):
    acc = lax.dot_general(
        x_ref[...],
        wt_ref[...],
        dimension_numbers=(((1,), (0,)), ((), ())),
        preferred_element_type=jnp.float32,
    )
    o_ref[...] = acc + b_ref[...]


@functools.cache
def _make_proj(B, H, V, bt):
    return pl.pallas_call(
        _proj_body,
        grid=(B // bt,),
        in_specs=[
            pl.BlockSpec((bt, H), lambda jb: (jb, 0)),
            pl.BlockSpec((H, V), lambda jb: (0, 0)),
            pl.BlockSpec((1, V), lambda jb: (0, 0)),
        ],
        out_specs=pl.BlockSpec((bt, V), lambda jb: (jb, 0)),
        out_shape=jax.ShapeDtypeStruct((B, V), jnp.float32),
        compiler_params=pltpu.CompilerParams(
            vmem_limit_bytes=100 * 1024 * 1024,
        ),
    )


def kernel(input_ids, embedding, lm_head_w, lm_head_b):
    B = input_ids.shape[0]
    V, H = embedding.shape
    x = _make_sc_gather(V, H, B)(embedding, input_ids.astype(jnp.int32))
    return _make_proj(B, H, V, 16)(x, lm_head_w.T, lm_head_b.reshape(1, V))


# speedup vs baseline: 1.0389x; 1.0389x over previous
import functools

import jax
import jax.numpy as jnp
from jax import lax
from jax.experimental import pallas as pl
from jax.experimental.pallas import tpu as pltpu
from jax.experimental.pallas import tpu_sc as plsc


@functools.cache
def _make_sc_gather(V, H, B):
    info = plsc.get_sparse_core_info()
    NC, NS = info.num_cores, info.num_subcores
    NW = NC * NS
    assert B % (8 * NW) == 0
    b_per_w = B // NW
    mesh = plsc.VectorSubcoreMesh(core_axis_name="c", subcore_axis_name="s")

    @functools.partial(
        pl.kernel,
        mesh=mesh,
        out_type=jax.ShapeDtypeStruct((B, H), jnp.float32),
        scratch_types=[
            pltpu.VMEM((b_per_w,), jnp.int32),
            pltpu.VMEM((b_per_w, H), jnp.float32),
            pltpu.SemaphoreType.DMA,
        ],
    )
    def gather_k(table_hbm, idx_hbm, out_hbm, idx_v, rows_v, sem):
        wid = lax.axis_index("s") * NC + lax.axis_index("c")
        base = wid * b_per_w
        pltpu.sync_copy(idx_hbm.at[pl.ds(base, b_per_w)], idx_v)
        for c in range(b_per_w // 16):
            chunk = idx_v[pl.ds(c * 16, 16)]
            for i in range(16):
                pltpu.async_copy(
                    table_hbm.at[pl.ds(chunk[i], 1)],
                    rows_v.at[pl.ds(c * 16 + i, 1)],
                    sem,
                )
        pltpu.make_async_copy(
            table_hbm.at[pl.ds(0, b_per_w)], rows_v, sem
        ).wait()
        pltpu.sync_copy(rows_v, out_hbm.at[pl.ds(base, b_per_w)])

    return gather_k


def _proj_body(x_ref, wt_ref, b_ref, o_ref):
    acc = lax.dot_general(
        x_ref[...],
        wt_ref[...],
        dimension_numbers=(((1,), (0,)), ((), ())),
        preferred_element_type=jnp.float32,
    )
    o_ref[...] = acc + b_ref[...]


@functools.cache
def _make_proj(B, H, V, bt):
    return pl.pallas_call(
        _proj_body,
        grid=(B // bt,),
        in_specs=[
            pl.BlockSpec((bt, H), lambda jb: (jb, 0)),
            pl.BlockSpec((H, V), lambda jb: (0, 0)),
            pl.BlockSpec((1, V), lambda jb: (0, 0)),
        ],
        out_specs=pl.BlockSpec((bt, V), lambda jb: (jb, 0)),
        out_shape=jax.ShapeDtypeStruct((B, V), jnp.float32),
        compiler_params=pltpu.CompilerParams(
            vmem_limit_bytes=100 * 1024 * 1024,
        ),
    )


def kernel(input_ids, embedding, lm_head_w, lm_head_b):
    B = input_ids.shape[0]
    V, H = embedding.shape
    x = _make_sc_gather(V, H, B)(embedding, input_ids.astype(jnp.int32))
    return _make_proj(B, H, V, 32)(x, lm_head_w.T, lm_head_b.reshape(1, V))
